# Initial kernel scaffold; baseline (speedup 1.0000x reference)
import jax
import jax.numpy as jnp
from jax.experimental import pallas as pl


def kernel(x, W, b):
    raise NotImplementedError("write your pallas kernel here")


# speedup vs baseline: 1.2179x; 1.2179x over previous
import functools

import jax
import jax.numpy as jnp
from jax import lax
from jax.experimental import pallas as pl
from jax.experimental.pallas import tpu as pltpu
from jax.experimental.pallas import tpu_sc as plsc

B, T, D = 4, 8192, 1024
C = T // 2
L = 16
NB = T // L
NC, NS = 2, 16
RPT = 2 * C // NS
GRP = 16
INFLIGHT = 4
WAVE = GRP * INFLIGHT

_I32 = jnp.int32
_U32 = jnp.uint32


def _logits_body(x_ref, w_ref, b_ref, o_ref):
    o_ref[...] = (
        jnp.dot(x_ref[...], w_ref[...], preferred_element_type=jnp.float32)
        + b_ref[0, 0]
    )


def _compute_logits(x2, wt, b):
    BLK = 2048
    return pl.pallas_call(
        _logits_body,
        grid=(x2.shape[0] // BLK,),
        in_specs=[
            pl.BlockSpec((BLK, D), lambda i: (i, 0)),
            pl.BlockSpec((D, 1), lambda i: (0, 0)),
            pl.BlockSpec((1, 1), lambda i: (0, 0), memory_space=pltpu.SMEM),
        ],
        out_specs=pl.BlockSpec((BLK, 1), lambda i: (i, 0)),
        out_shape=jax.ShapeDtypeStruct((x2.shape[0], 1), jnp.float32),
    )(x2, wt, b)


def _ckeys_pass(scores_v, ckeys_v):
    def body(i, _):
        s = scores_v[pl.ds(i * L, L)]
        u = lax.bitcast_convert_type(s, _U32)
        neg = (u >> 31) == 1
        key = jnp.where(neg, ~u, u | _U32(0x80000000))
        ckeys_v[pl.ds(i * L, L)] = ~key
        return 0

    lax.fori_loop(0, NB, body, 0)


def _hist_pass(ckeys_v, hist_v, nbins, shift, prefix_shift, prefix, target):
    nv = nbins // L
    zeros = jnp.zeros((L,), _I32)
    ones = jnp.ones((L,), _I32)

    def clr(j, _):
        hist_v[pl.ds(j * L, L)] = zeros
        return 0

    lax.fori_loop(0, nv, clr, 0)

    def acc(i, _):
        ck = ckeys_v[pl.ds(i * L, L)]
        binv = ((ck >> shift) & _U32(nbins - 1)).astype(_I32)
        if prefix_shift >= 32:
            plsc.addupdate_scatter(hist_v, [binv], ones)
        else:
            m = (ck >> prefix_shift) == prefix
            plsc.addupdate_scatter(hist_v, [binv], ones, mask=m)
        return 0

    lax.fori_loop(0, NB, acc, 0)

    lane = lax.iota(_I32, L)

    def scan(j, carry):
        run, cb, bstar = carry
        h = hist_v[pl.ds(j * L, L)]
        incl = plsc.cumsum(h) + run
        run2 = jnp.max(incl)
        cb2 = jnp.maximum(cb, jnp.max(jnp.where(incl < target, incl, 0)))
        binidx = lane + j * L
        hit = jnp.where(incl >= target, binidx, nbins)
        bstar2 = jnp.minimum(bstar, jnp.min(hit))
        return run2, cb2, bstar2

    _, cb, bstar = lax.fori_loop(
        0, nv, scan, (_I32(0), _I32(0), _I32(nbins)))
    return bstar, cb


def _compact_pass(ckeys_v, outidx_v, tau, m_ties):
    lane = lax.iota(_I32, L)

    def body(i, carry):
        pos, tiecnt = carry
        ck = ckeys_v[pl.ds(i * L, L)]
        idxv = lane + i * L
        strict = ck < tau
        tie = ck == tau
        tiei = tie.astype(_I32)
        tcum = plsc.cumsum(tiei)
        tie_sel = tie & ((tiecnt + tcum - 1) < m_ties)
        maskv = strict | tie_sel
        mi = maskv.astype(_I32)
        mcum = plsc.cumsum(mi)
        posv = pos + mcum - mi
        plsc.store_scatter(outidx_v, [posv], idxv, mask=maskv)
        return pos + jnp.sum(mi), tiecnt + jnp.sum(tiei)

    lax.fori_loop(0, NB, body, (_I32(0), _I32(0)))


def _sc_body(scores_hbm, x2_hbm, idx_hbm, sel_hbm,
             scores_v, ckeys_v, hist_v, outidx_v, idxg_v, rows_v,
             sp_idx, sem):
    c = lax.axis_index("c")
    s = lax.axis_index("s")

    @pl.when(s < 2)
    def _select():
        b = 2 * c + s
        pltpu.sync_copy(scores_hbm.at[b], scores_v)
        _ckeys_pass(scores_v, ckeys_v)

        b0, cb0 = _hist_pass(ckeys_v, hist_v, 2048, 21, 32, _U32(0), C)
        r1 = C - cb0
        p0 = b0.astype(_U32)
        b1, cb1 = _hist_pass(ckeys_v, hist_v, 2048, 10, 21, p0, r1)
        r2 = r1 - cb1
        p01 = (p0 << 11) | b1.astype(_U32)
        b2, cb2 = _hist_pass(ckeys_v, hist_v, 1024, 0, 10, p01, r2)
        tau = (p01 << 10) | b2.astype(_U32)
        m_ties = r2 - cb2

        _compact_pass(ckeys_v, outidx_v, tau, m_ties)
        pltpu.sync_copy(outidx_v, idx_hbm.at[b])
        pltpu.sync_copy(outidx_v, sp_idx.at[s])

    plsc.subcore_barrier()

    lb = s // 8
    b = 2 * c + lb
    r0 = (s % 8) * RPT
    pltpu.sync_copy(sp_idx.at[lb, pl.ds(r0, RPT)], idxg_v)
    out_base = b * C + r0

    def wave(w, _):
        descs = []
        for g in range(INFLIGHT):
            iv = idxg_v[pl.ds(w * WAVE + g * GRP, GRP)] + b * T
            descs.append(
                pltpu.async_copy(x2_hbm.at[iv],
                                 rows_v.at[pl.ds(g * GRP, GRP)], sem))
        for d in descs:
            d.wait()
        pltpu.sync_copy(rows_v, sel_hbm.at[pl.ds(out_base + w * WAVE, WAVE)])
        return 0

    lax.fori_loop(0, RPT // WAVE, wave, 0)


@functools.partial(
    pl.kernel,
    out_type=(
        jax.ShapeDtypeStruct((B, C), _I32),
        jax.ShapeDtypeStruct((B * C, D), jnp.float32),
    ),
    mesh=plsc.VectorSubcoreMesh(
        core_axis_name="c", subcore_axis_name="s",
        num_cores=NC, num_subcores=NS,
    ),
    compiler_params=pltpu.CompilerParams(needs_layout_passes=False),
    scratch_types=[
        pltpu.VMEM((T,), jnp.float32),
        pltpu.VMEM((T,), _U32),
        pltpu.VMEM((2048,), _I32),
        pltpu.VMEM((C,), _I32),
        pltpu.VMEM((RPT,), _I32),
        pltpu.VMEM((WAVE, D), jnp.float32),
        pltpu.VMEM_SHARED((2, C), _I32),
        pltpu.SemaphoreType.DMA,
    ],
)
def _sc_select_gather(scores_hbm, x2_hbm, idx_hbm, sel_hbm, *scratch):
    _sc_body(scores_hbm, x2_hbm, idx_hbm, sel_hbm, *scratch)


def kernel(x, W, b):
    x2 = x.reshape(B * T, D)
    logits2 = _compute_logits(x2, W.reshape(D, 1), b.reshape(1, 1))
    router_logits = logits2.reshape(B, T, 1)
    scores = logits2.reshape(B, T)
    topk_idx, sel2 = _sc_select_gather(scores, x2)
    return sel2.reshape(B, C, D), topk_idx, router_logits


# speedup vs baseline: 1.2465x; 1.0235x over previous
import functools

import jax
import jax.numpy as jnp
from jax import lax
from jax.experimental import pallas as pl
from jax.experimental.pallas import tpu as pltpu
from jax.experimental.pallas import tpu_sc as plsc

B, T, D = 4, 8192, 1024
C = T // 2
L = 16
NB = T // L
NC, NS = 2, 16
RPT = 2 * C // NS
GW = 32
NBUF = 3
NGW = RPT // GW

_I32 = jnp.int32
_U32 = jnp.uint32


def _logits_body(x_ref, w_ref, b_ref, o_ref):
    o_ref[...] = (
        jnp.dot(x_ref[...], w_ref[...], preferred_element_type=jnp.float32)
        + b_ref[0, 0]
    )


def _compute_logits(x2, wt, b):
    BLK = 4096
    nblk = x2.shape[0] // BLK
    out = pl.pallas_call(
        _logits_body,
        grid=(nblk,),
        in_specs=[
            pl.BlockSpec((BLK, D), lambda i: (i, 0)),
            pl.BlockSpec((D, 1), lambda i: (0, 0)),
            pl.BlockSpec((1, 1), lambda i: (0, 0), memory_space=pltpu.SMEM),
        ],
        out_specs=pl.BlockSpec((BLK, 1), lambda i: (i, 0)),
        out_shape=jax.ShapeDtypeStruct((x2.shape[0], 1), jnp.float32),
    )(x2, wt, b)
    return out


def _ckeys_pass(scores_v, ckeys_v):
    def body(i, _):
        s = scores_v[pl.ds(i * L, L)]
        u = lax.bitcast_convert_type(s, _U32)
        neg = (u >> 31) == 1
        key = jnp.where(neg, ~u, u | _U32(0x80000000))
        ckeys_v[pl.ds(i * L, L)] = ~key
        return 0

    lax.fori_loop(0, NB, body, 0)


def _hist_pass(ckeys_v, hist_v, nbins, shift, prefix_shift, prefix, target):
    nv = nbins // L
    zeros = jnp.zeros((L,), _I32)
    ones = jnp.ones((L,), _I32)

    def clr(j, _):
        hist_v[pl.ds(j * L, L)] = zeros
        return 0

    lax.fori_loop(0, nv, clr, 0)

    def acc(i, _):
        ck = ckeys_v[pl.ds(i * L, L)]
        binv = ((ck >> shift) & _U32(nbins - 1)).astype(_I32)
        if prefix_shift >= 32:
            plsc.addupdate_scatter(hist_v, [binv], ones)
        else:
            m = (ck >> prefix_shift) == prefix
            plsc.addupdate_scatter(hist_v, [binv], ones, mask=m)
        return 0

    lax.fori_loop(0, NB, acc, 0)

    lane = lax.iota(_I32, L)

    def scan(j, carry):
        run, cb, bstar = carry
        h = hist_v[pl.ds(j * L, L)]
        incl = plsc.cumsum(h) + run
        run2 = jnp.max(incl)
        cb2 = jnp.maximum(cb, jnp.max(jnp.where(incl < target, incl, 0)))
        binidx = lane + j * L
        hit = jnp.where(incl >= target, binidx, nbins)
        bstar2 = jnp.minimum(bstar, jnp.min(hit))
        return run2, cb2, bstar2

    _, cb, bstar = lax.fori_loop(
        0, nv, scan, (_I32(0), _I32(0), _I32(nbins)))
    return bstar, cb


def _compact_pass(ckeys_v, outidx_v, tau, m_ties):
    lane = lax.iota(_I32, L)

    def body(i, carry):
        pos, tiecnt = carry
        ck = ckeys_v[pl.ds(i * L, L)]
        idxv = lane + i * L
        strict = ck < tau
        tie = ck == tau
        tiei = tie.astype(_I32)
        tcum = plsc.cumsum(tiei)
        tie_sel = tie & ((tiecnt + tcum - 1) < m_ties)
        maskv = strict | tie_sel
        mi = maskv.astype(_I32)
        mcum = plsc.cumsum(mi)
        posv = pos + mcum - mi
        plsc.store_scatter(outidx_v, [posv], idxv, mask=maskv)
        return pos + jnp.sum(mi), tiecnt + jnp.sum(tiei)

    lax.fori_loop(0, NB, body, (_I32(0), _I32(0)))


def _sc_body(scores_hbm, x2_hbm, idx_hbm, sel_hbm,
             scores_v, ckeys_v, hist_v, outidx_v, idxg_v, rows_v,
             sp_idx, sem, gsem, wsem):
    c = lax.axis_index("c")
    s = lax.axis_index("s")

    @pl.when(s < 2)
    def _select():
        b = 2 * c + s
        pltpu.sync_copy(scores_hbm.at[b], scores_v)
        _ckeys_pass(scores_v, ckeys_v)

        b0, cb0 = _hist_pass(ckeys_v, hist_v, 2048, 21, 32, _U32(0), C)
        r1 = C - cb0
        p0 = b0.astype(_U32)
        b1, cb1 = _hist_pass(ckeys_v, hist_v, 2048, 10, 21, p0, r1)
        r2 = r1 - cb1
        p01 = (p0 << 11) | b1.astype(_U32)
        b2, cb2 = _hist_pass(ckeys_v, hist_v, 1024, 0, 10, p01, r2)
        tau = (p01 << 10) | b2.astype(_U32)
        m_ties = r2 - cb2

        _compact_pass(ckeys_v, outidx_v, tau, m_ties)
        pltpu.sync_copy(outidx_v, idx_hbm.at[b])
        pltpu.sync_copy(outidx_v, sp_idx.at[s])

    plsc.subcore_barrier()

    lb = s // 8
    b = 2 * c + lb
    r0 = (s % 8) * RPT
    pltpu.sync_copy(sp_idx.at[lb, pl.ds(r0, RPT)], idxg_v)

    def addoff(k, _):
        idxg_v[pl.ds(k * L, L)] = idxg_v[pl.ds(k * L, L)] + b * T
        return 0

    lax.fori_loop(0, RPT // L, addoff, 0)
    out_base = b * C + r0

    def ga_start(g):
        return pltpu.async_copy(
            x2_hbm.at[idxg_v.at[pl.ds(g * GW, GW)]],
            rows_v.at[g % NBUF], gsem)

    def wr_start(g):
        return pltpu.async_copy(
            rows_v.at[g % NBUF],
            sel_hbm.at[pl.ds(out_base + g * GW, GW)], wsem)

    ga = {0: ga_start(0)}
    wr = {}
    for g in range(1, NGW):
        if g >= NBUF:
            wr[g - NBUF].wait()
        ga[g] = ga_start(g)
        ga[g - 1].wait()
        wr[g - 1] = wr_start(g - 1)
    ga[NGW - 1].wait()
    wr[NGW - 1] = wr_start(NGW - 1)
    for g in range(NGW - NBUF, NGW):
        wr[g].wait()


@functools.partial(
    pl.kernel,
    out_type=(
        jax.ShapeDtypeStruct((B, C), _I32),
        jax.ShapeDtypeStruct((B * C, D), jnp.float32),
    ),
    mesh=plsc.VectorSubcoreMesh(
        core_axis_name="c", subcore_axis_name="s",
        num_cores=NC, num_subcores=NS,
    ),
    compiler_params=pltpu.CompilerParams(needs_layout_passes=False),
    scratch_types=[
        pltpu.VMEM((T,), jnp.float32),
        pltpu.VMEM((T,), _U32),
        pltpu.VMEM((2048,), _I32),
        pltpu.VMEM((C,), _I32),
        pltpu.VMEM((RPT,), _I32),
        pltpu.VMEM((NBUF, GW, D), jnp.float32),
        pltpu.VMEM_SHARED((2, C), _I32),
        pltpu.SemaphoreType.DMA,
        pltpu.SemaphoreType.DMA,
        pltpu.SemaphoreType.DMA,
    ],
)
def _sc_select_gather(scores_hbm, x2_hbm, idx_hbm, sel_hbm, *scratch):
    _sc_body(scores_hbm, x2_hbm, idx_hbm, sel_hbm, *scratch)


def kernel(x, W, b):
    x2 = x.reshape(B * T, D)
    logits2 = _compute_logits(x2, W.reshape(D, 1), b.reshape(1, 1))
    router_logits = logits2.reshape(B, T, 1)
    scores = logits2.reshape(B, T)
    topk_idx, sel2 = _sc_select_gather(scores, x2)
    return sel2.reshape(B, C, D), topk_idx, router_logits


# speedup vs baseline: 1.2471x; 1.0005x over previous
import functools

import jax
import jax.numpy as jnp
from jax import lax
from jax.experimental import pallas as pl
from jax.experimental.pallas import tpu as pltpu
from jax.experimental.pallas import tpu_sc as plsc

B, T, D = 4, 8192, 1024
C = T // 2
L = 16
NB = T // L
NC, NS = 2, 16
RPT = 2 * C // NS
GW = 32
NBUF = 3
NGW = RPT // GW

_I32 = jnp.int32
_U32 = jnp.uint32


def _logits_body(x_ref, w_ref, b_ref, o_ref):
    o_ref[...] = (
        jnp.dot(x_ref[...], w_ref[...], preferred_element_type=jnp.float32)
        + b_ref[0, 0]
    )


def _compute_logits(x2, wt, b):
    BLK = 4096
    nblk = x2.shape[0] // BLK
    return pl.pallas_call(
        _logits_body,
        grid=(nblk,),
        in_specs=[
            pl.BlockSpec((BLK, D), lambda i: (i, 0)),
            pl.BlockSpec((D, 1), lambda i: (0, 0)),
            pl.BlockSpec((1, 1), lambda i: (0, 0), memory_space=pltpu.SMEM),
        ],
        out_specs=pl.BlockSpec((BLK, 1), lambda i: (i, 0)),
        out_shape=jax.ShapeDtypeStruct((x2.shape[0], 1), jnp.float32),
    )(x2, wt, b)


def _ckeys_pass(scores_v, ckeys_v):
    def body(i, _):
        s = scores_v[pl.ds(i * L, L)]
        u = lax.bitcast_convert_type(s, _U32)
        neg = (u >> 31) == 1
        key = jnp.where(neg, ~u, u | _U32(0x80000000))
        ckeys_v[pl.ds(i * L, L)] = ~key
        return 0

    lax.fori_loop(0, NB, body, 0)


def _hist_pass(ckeys_v, hist_v, nbins, shift, prefix_shift, prefix, target):
    nv = nbins // L
    zeros = jnp.zeros((L,), _I32)
    ones = jnp.ones((L,), _I32)

    def clr(j, _):
        hist_v[pl.ds(j * L, L)] = zeros
        return 0

    lax.fori_loop(0, nv, clr, 0)

    def acc(i, _):
        ck = ckeys_v[pl.ds(i * L, L)]
        binv = ((ck >> shift) & _U32(nbins - 1)).astype(_I32)
        if prefix_shift >= 32:
            plsc.addupdate_scatter(hist_v, [binv], ones)
        else:
            m = (ck >> prefix_shift) == prefix
            plsc.addupdate_scatter(hist_v, [binv], ones, mask=m)
        return 0

    lax.fori_loop(0, NB, acc, 0)

    lane = lax.iota(_I32, L)

    def scan(j, carry):
        run, cb, bstar = carry
        h = hist_v[pl.ds(j * L, L)]
        incl = plsc.cumsum(h) + run
        run2 = jnp.max(incl)
        cb2 = jnp.maximum(cb, jnp.max(jnp.where(incl < target, incl, 0)))
        binidx = lane + j * L
        hit = jnp.where(incl >= target, binidx, nbins)
        bstar2 = jnp.minimum(bstar, jnp.min(hit))
        return run2, cb2, bstar2

    _, cb, bstar = lax.fori_loop(
        0, nv, scan, (_I32(0), _I32(0), _I32(nbins)))
    return bstar, cb


def _compact_pass(ckeys_v, outidx_v, tau, m_ties):
    lane = lax.iota(_I32, L)

    def body(i, carry):
        pos, tiecnt = carry
        ck = ckeys_v[pl.ds(i * L, L)]
        idxv = lane + i * L
        strict = ck < tau
        tie = ck == tau
        tiei = tie.astype(_I32)
        tcum = plsc.cumsum(tiei)
        tie_sel = tie & ((tiecnt + tcum - 1) < m_ties)
        maskv = strict | tie_sel
        mi = maskv.astype(_I32)
        mcum = plsc.cumsum(mi)
        posv = pos + mcum - mi
        plsc.store_scatter(outidx_v, [posv], idxv, mask=maskv)
        return pos + jnp.sum(mi), tiecnt + jnp.sum(tiei)

    lax.fori_loop(0, NB, body, (_I32(0), _I32(0)))


def _sc_body(scores_hbm, x2_hbm, idx_hbm, sel_hbm,
             scores_v, ckeys_v, hist_v, outidx_v, idxg_v, rows_v,
             sp_idx, gsem, wsem):
    c = lax.axis_index("c")
    s = lax.axis_index("s")

    @pl.when(s < 2)
    def _select():
        b = 2 * c + s
        pltpu.sync_copy(scores_hbm.at[b], scores_v)
        _ckeys_pass(scores_v, ckeys_v)

        b0, cb0 = _hist_pass(ckeys_v, hist_v, 2048, 21, 32, _U32(0), C)
        r1 = C - cb0
        p0 = b0.astype(_U32)
        b1, cb1 = _hist_pass(ckeys_v, hist_v, 2048, 10, 21, p0, r1)
        r2 = r1 - cb1
        p01 = (p0 << 11) | b1.astype(_U32)
        b2, cb2 = _hist_pass(ckeys_v, hist_v, 1024, 0, 10, p01, r2)
        tau = (p01 << 10) | b2.astype(_U32)
        m_ties = r2 - cb2

        _compact_pass(ckeys_v, outidx_v, tau, m_ties)
        pltpu.sync_copy(outidx_v, idx_hbm.at[b])
        pltpu.sync_copy(outidx_v, sp_idx.at[s])

    plsc.subcore_barrier()

    lb = s // 8
    b = 2 * c + lb
    r0 = (s % 8) * RPT
    pltpu.sync_copy(sp_idx.at[lb, pl.ds(r0, RPT)], idxg_v)

    def addoff(k, _):
        idxg_v[pl.ds(k * L, L)] = idxg_v[pl.ds(k * L, L)] + b * T
        return 0

    lax.fori_loop(0, RPT // L, addoff, 0)
    out_base = b * C + r0

    def ga_start(g):
        return pltpu.async_copy(
            x2_hbm.at[idxg_v.at[pl.ds(g * GW, GW)]],
            rows_v.at[g % NBUF], gsem)

    def wr_start(g):
        return pltpu.async_copy(
            rows_v.at[g % NBUF],
            sel_hbm.at[pl.ds(out_base + g * GW, GW)], wsem)

    ga = {0: ga_start(0)}
    wr = {}
    for g in range(1, NGW):
        if g >= NBUF:
            wr[g - NBUF].wait()
        ga[g] = ga_start(g)
        ga[g - 1].wait()
        wr[g - 1] = wr_start(g - 1)
    ga[NGW - 1].wait()
    wr[NGW - 1] = wr_start(NGW - 1)
    for g in range(NGW - NBUF, NGW):
        wr[g].wait()


@functools.partial(
    pl.kernel,
    out_type=(
        jax.ShapeDtypeStruct((B, C), _I32),
        jax.ShapeDtypeStruct((B * C, D), jnp.float32),
    ),
    mesh=plsc.VectorSubcoreMesh(
        core_axis_name="c", subcore_axis_name="s",
        num_cores=NC, num_subcores=NS,
    ),
    compiler_params=pltpu.CompilerParams(needs_layout_passes=False),
    scratch_types=[
        pltpu.VMEM((T,), jnp.float32),
        pltpu.VMEM((T,), _U32),
        pltpu.VMEM((2048,), _I32),
        pltpu.VMEM((C,), _I32),
        pltpu.VMEM((RPT,), _I32),
        pltpu.VMEM((NBUF, GW, D), jnp.float32),
        pltpu.VMEM_SHARED((2, C), _I32),
        pltpu.SemaphoreType.DMA,
        pltpu.SemaphoreType.DMA,
    ],
)
def _sc_select_gather(scores_hbm, x2_hbm, idx_hbm, sel_hbm, *scratch):
    _sc_body(scores_hbm, x2_hbm, idx_hbm, sel_hbm, *scratch)


def kernel(x, W, b):
    x2 = x.reshape(B * T, D)
    logits2 = _compute_logits(x2, W.reshape(D, 1), b.reshape(1, 1))
    router_logits = logits2.reshape(B, T, 1)
    scores = logits2.reshape(B, T)
    topk_idx, sel2 = _sc_select_gather(scores, x2)
    return sel2.reshape(B, C, D), topk_idx, router_logits
